# baseline (speedup 1.0000x reference)
import functools

import jax
import jax.numpy as jnp
from jax.experimental import pallas as pl
from jax.experimental.pallas import tpu as pltpu

E = 8
K = 2
D = 768
DFF = 768
OUT = 768
TOK = 2048

BT = 2048


def _fold_w1_body(w1_ref, out_ref):
    out_ref[0] = w1_ref[0, :D, :] + w1_ref[0, D:, :]


def _fold_w1(W1):
    return pl.pallas_call(
        _fold_w1_body,
        grid=(E,),
        in_specs=[pl.BlockSpec((1, 2 * D, DFF), lambda e: (e, 0, 0))],
        out_specs=pl.BlockSpec((1, D, DFF), lambda e: (e, 0, 0)),
        out_shape=jax.ShapeDtypeStruct((E, D, DFF), jnp.float32),
    )(W1)


def _moe_body(x_ref, wg_ref, bg_ref, w1_ref, b1_ref, w2_ref, b2_ref, out_ref):
    e = pl.program_id(1)
    x = x_ref[...]

    logits = jnp.dot(x, wg_ref[...], preferred_element_type=jnp.float32)
    logits = logits + bg_ref[0]
    lane = jax.lax.broadcasted_iota(jnp.int32, (BT, E), 1)
    m1 = jnp.max(logits, axis=1, keepdims=True)
    i1 = jnp.min(jnp.where(logits == m1, lane, E), axis=1, keepdims=True)
    l2 = jnp.where(lane == i1, -jnp.inf, logits)
    m2 = jnp.max(l2, axis=1, keepdims=True)
    i2 = jnp.min(jnp.where(l2 == m2, lane, E), axis=1, keepdims=True)
    t = jnp.exp(m2 - m1)
    w_first = 1.0 / (1.0 + t)
    w_second = 1.0 - w_first
    mw = jnp.where(i1 == e, w_first, 0.0) + jnp.where(i2 == e, w_second, 0.0)

    xb = x.astype(jnp.bfloat16)
    h = jnp.maximum(
        jnp.dot(xb, w1_ref[0].astype(jnp.bfloat16),
                preferred_element_type=jnp.float32) + b1_ref[0, 0],
        0.0,
    )
    y = jnp.dot(h.astype(jnp.bfloat16), w2_ref[0].astype(jnp.bfloat16),
                preferred_element_type=jnp.float32) + b2_ref[0, 0]
    contrib = mw * y

    @pl.when(e == 0)
    def _():
        out_ref[...] = contrib

    @pl.when(e != 0)
    def _():
        out_ref[...] += contrib


@jax.jit
def kernel(inputs, Wg, bg, W1, b1, W2, b2):
    W1f = _fold_w1(W1)
    bg2 = bg.reshape(1, E)
    b1r = b1.reshape(E, 1, DFF)
    b2r = b2.reshape(E, 1, OUT)
    nt = TOK // BT
    out = pl.pallas_call(
        _moe_body,
        grid=(nt, E),
        in_specs=[
            pl.BlockSpec((BT, D), lambda t, e: (t, 0)),
            pl.BlockSpec((D, E), lambda t, e: (0, 0)),
            pl.BlockSpec((1, E), lambda t, e: (0, 0)),
            pl.BlockSpec((1, D, DFF), lambda t, e: (e, 0, 0)),
            pl.BlockSpec((1, 1, DFF), lambda t, e: (e, 0, 0)),
            pl.BlockSpec((1, DFF, OUT), lambda t, e: (e, 0, 0)),
            pl.BlockSpec((1, 1, OUT), lambda t, e: (e, 0, 0)),
        ],
        out_specs=pl.BlockSpec((BT, OUT), lambda t, e: (t, 0)),
        out_shape=jax.ShapeDtypeStruct((TOK, OUT), jnp.float32),
        compiler_params=pltpu.CompilerParams(
            dimension_semantics=("parallel", "arbitrary"),
        ),
    )(inputs, Wg, bg2, W1f, b1r, W2, b2r)
    return out


# speedup vs baseline: 1.2861x; 1.2861x over previous
import functools

import jax
import jax.numpy as jnp
from jax.experimental import pallas as pl
from jax.experimental.pallas import tpu as pltpu

E = 8
K = 2
D = 768
DFF = 768
OUT = 768
TOK = 2048

BT = 2048


def _moe_body(x_ref, wg_ref, bg_ref, w1_ref, b1_ref, w2_ref, b2_ref, out_ref):
    e = pl.program_id(1)
    x = x_ref[...]

    logits = jnp.dot(x, wg_ref[...], preferred_element_type=jnp.float32)
    logits = logits + bg_ref[0]
    lane = jax.lax.broadcasted_iota(jnp.int32, (BT, E), 1)
    m1 = jnp.max(logits, axis=1, keepdims=True)
    i1 = jnp.min(jnp.where(logits == m1, lane, E), axis=1, keepdims=True)
    l2 = jnp.where(lane == i1, -jnp.inf, logits)
    m2 = jnp.max(l2, axis=1, keepdims=True)
    i2 = jnp.min(jnp.where(l2 == m2, lane, E), axis=1, keepdims=True)
    t = jnp.exp(m2 - m1)
    w_first = 1.0 / (1.0 + t)
    w_second = 1.0 - w_first
    mw = jnp.where(i1 == e, w_first, 0.0) + jnp.where(i2 == e, w_second, 0.0)

    w1f = w1_ref[0, :D, :] + w1_ref[0, D:, :]
    h = jnp.maximum(
        jnp.dot(x, w1f, preferred_element_type=jnp.float32) + b1_ref[0, 0],
        0.0,
    )
    y = jnp.dot(h, w2_ref[0], preferred_element_type=jnp.float32) + b2_ref[0, 0]
    contrib = mw * y

    @pl.when(e == 0)
    def _():
        out_ref[...] = contrib

    @pl.when(e != 0)
    def _():
        out_ref[...] += contrib


@jax.jit
def kernel(inputs, Wg, bg, W1, b1, W2, b2):
    bg2 = bg.reshape(1, E)
    b1r = b1.reshape(E, 1, DFF)
    b2r = b2.reshape(E, 1, OUT)
    nt = TOK // BT
    out = pl.pallas_call(
        _moe_body,
        grid=(nt, E),
        in_specs=[
            pl.BlockSpec((BT, D), lambda t, e: (t, 0)),
            pl.BlockSpec((D, E), lambda t, e: (0, 0)),
            pl.BlockSpec((1, E), lambda t, e: (0, 0)),
            pl.BlockSpec((1, 2 * D, DFF), lambda t, e: (e, 0, 0)),
            pl.BlockSpec((1, 1, DFF), lambda t, e: (e, 0, 0)),
            pl.BlockSpec((1, DFF, OUT), lambda t, e: (e, 0, 0)),
            pl.BlockSpec((1, 1, OUT), lambda t, e: (e, 0, 0)),
        ],
        out_specs=pl.BlockSpec((BT, OUT), lambda t, e: (t, 0)),
        out_shape=jax.ShapeDtypeStruct((TOK, OUT), jnp.float32),
        compiler_params=pltpu.CompilerParams(
            dimension_semantics=("parallel", "arbitrary"),
        ),
    )(inputs, Wg, bg2, W1, b1r, W2, b2r)
    return out
